# Initial kernel scaffold; baseline (speedup 1.0000x reference)
import jax
import jax.numpy as jnp
from jax.experimental import pallas as pl


def kernel(locs, init_embeddings, W, b):
    raise NotImplementedError("write your pallas kernel here")


# speedup vs baseline: 1.3226x; 1.3226x over previous
import functools
import jax
import jax.numpy as jnp
from jax.experimental import pallas as pl

B_, N_, K_, D_ = 16, 1000, 32, 128
N2 = 1024
RBLK = 200
EBLK = 512
BIG = 1e30


def _topk_body(xq_ref, yq_ref, xa_ref, ya_ref, vals_ref, idx_ref):
    b = pl.program_id(0)
    r = pl.program_id(1)
    xq = xq_ref[0]
    yq = yq_ref[0]
    xa = xa_ref[0]
    ya = ya_ref[0]
    dx = xq - xa
    dy = yq - ya
    d2 = dx * dx + dy * dy
    colio = jax.lax.broadcasted_iota(jnp.int32, (RBLK, N2), 1)
    rowg = r * RBLK + jax.lax.broadcasted_iota(jnp.int32, (RBLK, N2), 0)
    d2 = jnp.where((colio == rowg) | (colio >= N_), BIG, d2)

    kio = jax.lax.broadcasted_iota(jnp.int32, (RBLK, K_), 1)
    valsacc = jnp.zeros((RBLK, K_), jnp.float32)
    idxacc = jnp.zeros((RBLK, K_), jnp.int32)

    def step(k, carry):
        d2c, va, ia = carry
        m = jnp.min(d2c, axis=1, keepdims=True)
        am = jnp.min(jnp.where(d2c == m, colio, N2 + 1), axis=1,
                     keepdims=True)
        va = jnp.where(kio == k, m, va)
        ia = jnp.where(kio == k, am, ia)
        d2c = jnp.where(colio == am, BIG, d2c)
        return d2c, va, ia

    d2, valsacc, idxacc = jax.lax.fori_loop(0, K_, step, (d2, valsacc, idxacc))
    vals_ref[0] = jnp.sqrt(valsacc + 1e-12)
    idx_ref[0] = idxacc + b * N_


def _edge_body(va_ref, w_ref, bias_ref, emb_ref, src_ref):
    g = pl.program_id(0)
    v = va_ref[...]
    emb_ref[...] = v * w_ref[...] + bias_ref[...]
    rio = jax.lax.broadcasted_iota(jnp.int32, (EBLK, 1), 0)
    src_ref[...] = jax.lax.shift_right_logical(g * EBLK + rio, 5)


@jax.jit
def kernel(locs, init_embeddings, W, b):
    x = init_embeddings.reshape(B_ * N_, D_)

    xq = locs[:, :, 0:1]
    yq = locs[:, :, 1:2]
    xa = jnp.pad(locs[:, :, 0], ((0, 0), (0, N2 - N_)))[:, None, :]
    ya = jnp.pad(locs[:, :, 1], ((0, 0), (0, N2 - N_)))[:, None, :]

    vals, dst = pl.pallas_call(
        _topk_body,
        grid=(B_, N_ // RBLK),
        in_specs=[
            pl.BlockSpec((1, RBLK, 1), lambda b_, r: (b_, r, 0)),
            pl.BlockSpec((1, RBLK, 1), lambda b_, r: (b_, r, 0)),
            pl.BlockSpec((1, 1, N2), lambda b_, r: (b_, 0, 0)),
            pl.BlockSpec((1, 1, N2), lambda b_, r: (b_, 0, 0)),
        ],
        out_specs=[
            pl.BlockSpec((1, RBLK, K_), lambda b_, r: (b_, r, 0)),
            pl.BlockSpec((1, RBLK, K_), lambda b_, r: (b_, r, 0)),
        ],
        out_shape=[
            jax.ShapeDtypeStruct((B_, N_, K_), jnp.float32),
            jax.ShapeDtypeStruct((B_, N_, K_), jnp.int32),
        ],
    )(xq, yq, xa, ya)

    E = B_ * N_ * K_
    va = vals.reshape(E, 1)
    emb, src = pl.pallas_call(
        _edge_body,
        grid=(E // EBLK,),
        in_specs=[
            pl.BlockSpec((EBLK, 1), lambda g: (g, 0)),
            pl.BlockSpec((1, D_), lambda g: (0, 0)),
            pl.BlockSpec((1, D_), lambda g: (0, 0)),
        ],
        out_specs=[
            pl.BlockSpec((EBLK, D_), lambda g: (g, 0)),
            pl.BlockSpec((EBLK, 1), lambda g: (g, 0)),
        ],
        out_shape=[
            jax.ShapeDtypeStruct((E, D_), jnp.float32),
            jax.ShapeDtypeStruct((E, 1), jnp.int32),
        ],
    )(va, W, b.reshape(1, D_))

    edge_index = jnp.stack([src.reshape(-1), dst.reshape(-1)], axis=0)
    return x, edge_index, emb


# speedup vs baseline: 1.3284x; 1.0044x over previous
import functools
import jax
import jax.numpy as jnp
from jax import lax
from jax.experimental import pallas as pl
from jax.experimental.pallas import tpu as pltpu
from jax.experimental.pallas import tpu_sc as plsc

B_, N_, K_, D_ = 16, 1000, 32, 128
N2 = 1024
NW = 32
RPW = (B_ * N_) // NW
EBLK = 512
BIG = 1e30
CAP = 192
TRIG = CAP - 32


def _merge16(A, Ai, Bv, Bi, C, Ci):
    Cr = lax.rev(C, dimensions=(0,))
    Cri = lax.rev(Ci, dimensions=(0,))
    m = Bv <= Cr
    L = jnp.where(m, Bv, Cr)
    Li = jnp.where(m, Bi, Cri)
    Ls, Lsi = plsc.sort_key_val(L, Li)
    Lr = lax.rev(Ls, dimensions=(0,))
    Lri = lax.rev(Lsi, dimensions=(0,))
    m2 = A <= Lr
    A2 = jnp.where(m2, A, Lr)
    A2i = jnp.where(m2, Ai, Lri)
    B2 = jnp.where(m2, Lr, A)
    B2i = jnp.where(m2, Lri, Ai)
    A3, A3i = plsc.sort_key_val(A2, A2i)
    B3, B3i = plsc.sort_key_val(B2, B2i)
    return A3, A3i, B3, B3i


def _drain(bufv, bufi, A, Ai, Bv, Bi, ptr):
    nch = (ptr + 15) // 16

    def fold(cc, carry):
        A, Ai, Bv, Bi = carry
        v = bufv[pl.ds(cc * 16, 16)]
        vi = bufi[pl.ds(cc * 16, 16)]
        lanes = cc * 16 + lax.iota(jnp.int32, 16)
        v = jnp.where(lanes < ptr, v, BIG)
        vs, vsi = plsc.sort_key_val(v, vi)
        return _merge16(A, Ai, Bv, Bi, vs, vsi)

    A, Ai, Bv, Bi = lax.fori_loop(0, nch, fold, (A, Ai, Bv, Bi))
    thr = jnp.max(Bv)
    return A, Ai, Bv, Bi, thr, jnp.int32(0)


def _sc_body(xpad, ypad, vals_out, idx_out,
             xs_v, ys_v, bufv, bufi, vstage, istage, sem):
    wid = lax.axis_index("s") * 2 + lax.axis_index("c")
    row0 = wid * RPW
    g = row0 // N_
    lbase = row0 - g * N_

    pltpu.sync_copy(xpad.at[g], xs_v)
    pltpu.sync_copy(ypad.at[g], ys_v)

    def row_body(rl, _):
        i = lbase + rl
        xi = xs_v[pl.ds(i, 16)][0]
        yi = ys_v[pl.ds(i, 16)][0]
        A0 = jnp.full((16,), BIG, jnp.float32)
        I0 = jnp.zeros((16,), jnp.int32)

        def chunk(c, carry):
            A, Ai, Bv, Bi, thr, ptr = carry
            xs = xs_v[pl.ds(c * 16, 16)]
            ys = ys_v[pl.ds(c * 16, 16)]
            dx = xs - xi
            dy = ys - yi
            d2 = dx * dx + dy * dy
            cols = c * 16 + lax.iota(jnp.int32, 16)
            d2 = jnp.where((cols == i) | (cols >= N_), BIG, d2)
            m = d2 < thr
            mi = m.astype(jnp.int32)
            cum = plsc.cumsum(mi)
            slots = ptr + cum - mi
            plsc.store_scatter(bufv, [slots], d2, mask=m)
            plsc.store_scatter(bufi, [slots], cols, mask=m)
            ptr = ptr + jnp.max(cum)
            return lax.cond(
                ptr >= TRIG,
                lambda a, ai, b, bi, p: _drain(bufv, bufi, a, ai, b, bi, p),
                lambda a, ai, b, bi, p: (a, ai, b, bi, thr, p),
                A, Ai, Bv, Bi, ptr)

        carry = (A0, I0, A0, I0, jnp.float32(BIG), jnp.int32(0))
        A, Ai, Bv, Bi, thr, ptr = lax.fori_loop(0, N2 // 16, chunk, carry)
        A, Ai, Bv, Bi, thr, ptr = _drain(bufv, bufi, A, Ai, Bv, Bi, ptr)

        vstage[pl.ds(rl * 32, 16)] = A
        vstage[pl.ds(rl * 32 + 16, 16)] = Bv
        istage[pl.ds(rl * 32, 16)] = Ai + g * N_
        istage[pl.ds(rl * 32 + 16, 16)] = Bi + g * N_
        return 0

    lax.fori_loop(0, RPW, row_body, 0)
    pltpu.sync_copy(vstage, vals_out.at[pl.ds(row0 * K_, RPW * K_)])
    pltpu.sync_copy(istage, idx_out.at[pl.ds(row0 * K_, RPW * K_)])


_sc_topk = functools.partial(
    pl.kernel,
    out_type=[
        jax.ShapeDtypeStruct((B_ * N_ * K_,), jnp.float32),
        jax.ShapeDtypeStruct((B_ * N_ * K_,), jnp.int32),
    ],
    mesh=plsc.VectorSubcoreMesh(core_axis_name="c", subcore_axis_name="s"),
    compiler_params=pltpu.CompilerParams(needs_layout_passes=False),
    scratch_types=[
        pltpu.VMEM((N2,), jnp.float32),
        pltpu.VMEM((N2,), jnp.float32),
        pltpu.VMEM((CAP,), jnp.float32),
        pltpu.VMEM((CAP,), jnp.int32),
        pltpu.VMEM((RPW * K_,), jnp.float32),
        pltpu.VMEM((RPW * K_,), jnp.int32),
        pltpu.SemaphoreType.DMA,
    ],
)(_sc_body)


def _edge_body(va_ref, w_ref, bias_ref, emb_ref, src_ref):
    gg = pl.program_id(0)
    d = jnp.sqrt(va_ref[...] + 1e-12)
    emb_ref[...] = d * w_ref[...] + bias_ref[...]
    rio = lax.broadcasted_iota(jnp.int32, (EBLK, 1), 0)
    src_ref[...] = lax.shift_right_logical(gg * EBLK + rio, 5)


@jax.jit
def kernel(locs, init_embeddings, W, b):
    x = init_embeddings.reshape(B_ * N_, D_)

    xpad = jnp.pad(locs[:, :, 0], ((0, 0), (0, N2 - N_)))
    ypad = jnp.pad(locs[:, :, 1], ((0, 0), (0, N2 - N_)))

    d2vals, dst = _sc_topk(xpad, ypad)

    E = B_ * N_ * K_
    va = d2vals.reshape(E, 1)
    emb, src = pl.pallas_call(
        _edge_body,
        grid=(E // EBLK,),
        in_specs=[
            pl.BlockSpec((EBLK, 1), lambda gg: (gg, 0)),
            pl.BlockSpec((1, D_), lambda gg: (0, 0)),
            pl.BlockSpec((1, D_), lambda gg: (0, 0)),
        ],
        out_specs=[
            pl.BlockSpec((EBLK, D_), lambda gg: (gg, 0)),
            pl.BlockSpec((EBLK, 1), lambda gg: (gg, 0)),
        ],
        out_shape=[
            jax.ShapeDtypeStruct((E, D_), jnp.float32),
            jax.ShapeDtypeStruct((E, 1), jnp.int32),
        ],
    )(va, W, b.reshape(1, D_))

    edge_index = jnp.stack([src.reshape(-1), dst], axis=0)
    return x, edge_index, emb
